# baseline (speedup 1.0000x reference)
import functools

import jax
import jax.numpy as jnp
from jax import lax
from jax.experimental import pallas as pl
from jax.experimental.pallas import tpu as pltpu
from jax.experimental.pallas import tpu_sc as plsc

NC, NS, LANES = 2, 16, 16
NW = NC * NS
CHD = 1000
CHA = 400
WB = 1000
SCALE_UNROLL = 1


def _edge_split(nchunks_total, wid):
    base = nchunks_total // NW
    rem = nchunks_total - base * NW
    r0 = base * wid + jnp.minimum(wid, rem)
    nrows = base + jnp.where(wid < rem, 1, 0)
    return r0, nrows


def _deg_kernel_body(n, dst_hbm, ew_hbm, out_hbm, dst_v, ew_v, buf_v, deg_sh, dsem):
    c = lax.axis_index("c")
    s = lax.axis_index("s")
    wid = c * NS + s
    nwb = n // WB
    kd = dst_hbm.shape[1]

    @pl.when(s < nwb)
    def _():
        def zb(i, carry):
            buf_v[pl.ds(i * 16, 16)] = jnp.zeros((16,), jnp.float32)
            return carry
        lax.fori_loop(0, buf_v.shape[0] // 16, zb, 0)
        pltpu.sync_copy(buf_v.at[pl.ds(0, WB)],
                        deg_sh.at[pl.ds(pl.multiple_of(s * WB, 8), WB)])

    pltpu.sync_copy(dst_hbm.at[wid], dst_v)
    pltpu.sync_copy(ew_hbm.at[wid], ew_v)
    plsc.subcore_barrier()

    handles = [pltpu.async_copy(ew_v.at[i], deg_sh.at[dst_v.at[i]], dsem,
                                add=True)
               for i in range(kd)]
    for h in handles:
        h.wait()
    plsc.subcore_barrier()

    @pl.when(s < nwb)
    def _():
        src_off = pl.multiple_of(s * WB, 8)
        dst_off = pl.multiple_of(c * n + s * WB, 8)
        pltpu.sync_copy(deg_sh.at[pl.ds(src_off, WB)], buf_v.at[pl.ds(0, WB)])
        pltpu.sync_copy(buf_v.at[pl.ds(0, WB)], out_hbm.at[pl.ds(dst_off, WB)])


def _make_deg_kernel(n, kd):
    assert n % WB == 0 and (n // WB) <= NS
    mesh = plsc.VectorSubcoreMesh(core_axis_name="c", subcore_axis_name="s")
    return pl.kernel(
        functools.partial(_deg_kernel_body, n),
        out_type=jax.ShapeDtypeStruct((NC * n,), jnp.float32),
        mesh=mesh,
        compiler_params=pltpu.CompilerParams(use_tc_tiling_on_sc=False),
        scratch_types=[
            pltpu.VMEM((kd, CHD), jnp.int32),
            pltpu.VMEM((kd, CHD), jnp.float32),
            pltpu.VMEM((((WB + 15) // 16) * 16,), jnp.float32),
            pltpu.VMEM_SHARED((n,), jnp.float32),
            pltpu.SemaphoreType.DMA,
        ],
    )


def _agg_kernel_body(n, mp_hbm, src_hbm, dst_hbm, ew_hbm, out_hbm,
                     idx_v, dst_v, ew_v, rows_v, acc_sh, gsem, ssem):
    c = lax.axis_index("c")
    s = lax.axis_index("s")
    wid = c * NS + s
    hid = mp_hbm.shape[1]
    nq = hid // 16
    ka = src_hbm.shape[1]
    npt = n // NS
    row_chunks = []
    o = 0
    while o < npt:
        w = min(CHA, npt - o)
        row_chunks.append((o, w))
        o += w

    pltpu.sync_copy(src_hbm.at[wid], idx_v)
    pltpu.sync_copy(dst_hbm.at[wid], dst_v)
    pltpu.sync_copy(ew_hbm.at[wid], ew_v)

    def zb(j, carry):
        for q in range(nq):
            rows_v[0, j, pl.ds(q * 16, 16)] = jnp.zeros((16,), jnp.float32)
        return carry
    lax.fori_loop(0, CHA, zb, 0)
    for (o, w) in row_chunks:
        pltpu.sync_copy(rows_v.at[0, pl.ds(0, w)],
                        acc_sh.at[pl.ds(s * npt + o, w)])
    plsc.subcore_barrier()

    def gather(i, b):
        return pltpu.async_copy(mp_hbm.at[idx_v.at[i]], rows_v.at[b], gsem[b])

    def wait_gather(b):
        pltpu.make_async_copy(mp_hbm.at[idx_v.at[0]], rows_v.at[b],
                              gsem[b]).wait()

    def scatter(i, b):
        return pltpu.async_copy(rows_v.at[b], acc_sh.at[dst_v.at[i]],
                                ssem[b], add=True)

    def wait_scatter(b):
        pltpu.make_async_copy(rows_v.at[b], acc_sh.at[dst_v.at[0]],
                              ssem[b]).wait()

    def scale(b, i):
        @plsc.parallel_loop(0, CHA // 16, step=1, unroll=SCALE_UNROLL)
        def sbody(g):
            wv = ew_v[i, pl.ds(g * 16, 16)]
            for l in range(16):
                j = g * 16 + l
                wb = jnp.full((16,), wv[l], jnp.float32)
                for q in range(nq):
                    rows_v[b, j, pl.ds(q * 16, 16)] = (
                        rows_v[b, j, pl.ds(q * 16, 16)] * wb)

    npairs = ka // 2
    gather(0, 0)

    def pair(k, carry):
        i0 = 2 * k
        i1 = 2 * k + 1

        @pl.when(k > 0)
        def _():
            wait_scatter(1)
        gather(i1, 1)
        wait_gather(0)
        scale(0, i0)
        scatter(i0, 0)
        wait_gather(1)
        scale(1, i1)
        scatter(i1, 1)

        @pl.when(k < npairs - 1)
        def _():
            wait_scatter(0)
            gather(i0 + 2, 0)
        return carry
    lax.fori_loop(0, npairs, pair, 0)
    wait_scatter(0)
    wait_scatter(1)
    plsc.subcore_barrier()

    for (o, w) in row_chunks:
        pltpu.sync_copy(acc_sh.at[pl.ds(s * npt + o, w)],
                        out_hbm.at[c, pl.ds(s * npt + o, w)])


def _make_agg_kernel(n, hid, ka):
    assert n % NS == 0 and hid % 16 == 0
    mesh = plsc.VectorSubcoreMesh(core_axis_name="c", subcore_axis_name="s")
    return pl.kernel(
        functools.partial(_agg_kernel_body, n),
        out_type=jax.ShapeDtypeStruct((NC, n, hid), jnp.float32),
        mesh=mesh,
        compiler_params=pltpu.CompilerParams(use_tc_tiling_on_sc=False),
        scratch_types=[
            pltpu.VMEM((ka, CHA), jnp.int32),
            pltpu.VMEM((ka, CHA), jnp.int32),
            pltpu.VMEM((ka, CHA), jnp.float32),
            pltpu.VMEM((2, CHA, hid), jnp.float32),
            pltpu.VMEM_SHARED((n, hid), jnp.float32),
            [pltpu.SemaphoreType.DMA, pltpu.SemaphoreType.DMA],
            [pltpu.SemaphoreType.DMA, pltpu.SemaphoreType.DMA],
        ],
    )


def _dinv(degp_ref):
    deg = 1.0 + degp_ref[0, :] + degp_ref[1, :]
    return lax.rsqrt(deg)


def _tc_a_body(x_ref, wf_ref, bf_ref, wc1_ref, degp_ref, mp_ref):
    h = jnp.maximum(
        jnp.dot(x_ref[...], wf_ref[...], preferred_element_type=jnp.float32)
        + bf_ref[...][None, :], 0.0)
    dinv = _dinv(degp_ref)
    mp_ref[...] = dinv[:, None] * jnp.dot(
        h, wc1_ref[...], preferred_element_type=jnp.float32)


def _tc_b_body(aggp_ref, mp_ref, b_ref, w_next_ref, degp_ref, out_ref):
    dinv = _dinv(degp_ref)[:, None]
    h = jnp.maximum(
        dinv * (aggp_ref[0] + aggp_ref[1] + mp_ref[...]) + b_ref[...][None, :],
        0.0)
    out_ref[...] = dinv * jnp.dot(
        h, w_next_ref[...], preferred_element_type=jnp.float32)


def _tc_c_body(aggp_ref, mp_ref, b_ref, wout_ref, bout_ref, degp_ref, out_ref):
    dinv = _dinv(degp_ref)[:, None]
    h = jnp.maximum(
        dinv * (aggp_ref[0] + aggp_ref[1] + mp_ref[...]) + b_ref[...][None, :],
        0.0)
    logits = jnp.dot(h, wout_ref[...], preferred_element_type=jnp.float32) \
        + bout_ref[...][None, :]
    m = jnp.max(logits, axis=-1, keepdims=True)
    lse = m + jnp.log(jnp.sum(jnp.exp(logits - m), axis=-1, keepdims=True))
    out_ref[...] = logits - lse


def _tc_call(body, out_shape):
    return pl.pallas_call(body, out_shape=jax.ShapeDtypeStruct(out_shape, jnp.float32))


def kernel(x, edge_index, edge_weight, W_first, b_first, W_c1, b_c1,
           W_c2, b_c2, W_out, b_out):
    n, f_in = x.shape
    hid = W_c1.shape[0]
    e = edge_weight.shape[0]
    assert e % (CHD * NW) == 0 and e % (CHA * NW) == 0
    kd = e // (CHD * NW)
    ka = e // (CHA * NW)
    src3a = edge_index[0].reshape(NW, ka, CHA)
    dst3a = edge_index[1].reshape(NW, ka, CHA)
    ew3a = edge_weight.reshape(NW, ka, CHA)
    if ka % 2:
        ka += 1
        zi = jnp.zeros((NW, 1, CHA), jnp.int32)
        zf = jnp.zeros((NW, 1, CHA), jnp.float32)
        src3a = jnp.concatenate([src3a, zi], axis=1)
        dst3a = jnp.concatenate([dst3a, zi], axis=1)
        ew3a = jnp.concatenate([ew3a, zf], axis=1)
    dst3d = edge_index[1].reshape(NW, kd, CHD)
    ew3d = edge_weight.reshape(NW, kd, CHD)

    degf = _make_deg_kernel(n, kd)(dst3d, ew3d)
    degp = degf.reshape(NC, n)

    mp1 = _tc_call(_tc_a_body, (n, hid))(x, W_first, b_first, W_c1, degp)
    agg1 = _make_agg_kernel(n, hid, ka)(mp1, src3a, dst3a, ew3a)
    mp2 = _tc_call(_tc_b_body, (n, hid))(agg1, mp1, b_c1, W_c2, degp)
    agg2 = _make_agg_kernel(n, hid, ka)(mp2, src3a, dst3a, ew3a)
    out = _tc_call(_tc_c_body, (n, W_out.shape[1]))(
        agg2, mp2, b_c2, W_out, b_out, degp)
    return out


# speedup vs baseline: 2.8600x; 2.8600x over previous
import functools

import jax
import jax.numpy as jnp
from jax import lax
from jax.experimental import pallas as pl
from jax.experimental.pallas import tpu as pltpu
from jax.experimental.pallas import tpu_sc as plsc

NC, NS, LANES = 2, 16, 16
NW = NC * NS
CHD = 1000
CHA = 400
WB = 1000
SCALE_UNROLL = 1


def _edge_split(nchunks_total, wid):
    base = nchunks_total // NW
    rem = nchunks_total - base * NW
    r0 = base * wid + jnp.minimum(wid, rem)
    nrows = base + jnp.where(wid < rem, 1, 0)
    return r0, nrows


def _deg_kernel_body(n, dst_hbm, ew_hbm, out_hbm, dst_v, ew_v, buf_v, deg_sh, dsem):
    c = lax.axis_index("c")
    s = lax.axis_index("s")
    wid = c * NS + s
    nwb = n // WB
    kd = dst_hbm.shape[1]

    @pl.when(s < nwb)
    def _():
        def zb(i, carry):
            buf_v[pl.ds(i * 16, 16)] = jnp.zeros((16,), jnp.float32)
            return carry
        lax.fori_loop(0, buf_v.shape[0] // 16, zb, 0)
        pltpu.sync_copy(buf_v.at[pl.ds(0, WB)],
                        deg_sh.at[pl.ds(pl.multiple_of(s * WB, 8), WB)])

    pltpu.sync_copy(dst_hbm.at[wid], dst_v)
    pltpu.sync_copy(ew_hbm.at[wid], ew_v)
    plsc.subcore_barrier()

    handles = [pltpu.async_copy(ew_v.at[i], deg_sh.at[dst_v.at[i]], dsem,
                                add=True)
               for i in range(kd)]
    for h in handles:
        h.wait()
    plsc.subcore_barrier()

    @pl.when(s < nwb)
    def _():
        src_off = pl.multiple_of(s * WB, 8)
        dst_off = pl.multiple_of(c * n + s * WB, 8)
        pltpu.sync_copy(deg_sh.at[pl.ds(src_off, WB)], buf_v.at[pl.ds(0, WB)])
        pltpu.sync_copy(buf_v.at[pl.ds(0, WB)], out_hbm.at[pl.ds(dst_off, WB)])


def _make_deg_kernel(n, kd):
    assert n % WB == 0 and (n // WB) <= NS
    mesh = plsc.VectorSubcoreMesh(core_axis_name="c", subcore_axis_name="s")
    return pl.kernel(
        functools.partial(_deg_kernel_body, n),
        out_type=jax.ShapeDtypeStruct((NC * n,), jnp.float32),
        mesh=mesh,
        compiler_params=pltpu.CompilerParams(use_tc_tiling_on_sc=False),
        scratch_types=[
            pltpu.VMEM((kd, CHD), jnp.int32),
            pltpu.VMEM((kd, CHD), jnp.float32),
            pltpu.VMEM((((WB + 15) // 16) * 16,), jnp.float32),
            pltpu.VMEM_SHARED((n,), jnp.float32),
            pltpu.SemaphoreType.DMA,
        ],
    )


def _agg_kernel_body(n, mp_hbm, src_hbm, dst_hbm, ew_hbm, out_hbm,
                     idx_v, dst_v, ew_v, rows_v, acc_sh, gsem, ssem):
    c = lax.axis_index("c")
    s = lax.axis_index("s")
    wid = c * NS + s
    hid = mp_hbm.shape[1]
    nq = hid // 16
    ka = src_hbm.shape[1]
    npt = n // NS
    row_chunks = []
    o = 0
    while o < npt:
        w = min(CHA, npt - o)
        row_chunks.append((o, w))
        o += w

    pltpu.sync_copy(src_hbm.at[wid], idx_v)
    pltpu.sync_copy(dst_hbm.at[wid], dst_v)
    pltpu.sync_copy(ew_hbm.at[wid], ew_v)

    def zb(j, carry):
        for q in range(nq):
            rows_v[0, j, pl.ds(q * 16, 16)] = jnp.zeros((16,), jnp.float32)
        return carry
    lax.fori_loop(0, CHA, zb, 0)
    for (o, w) in row_chunks:
        pltpu.sync_copy(rows_v.at[0, pl.ds(0, w)],
                        acc_sh.at[pl.ds(s * npt + o, w)])
    plsc.subcore_barrier()

    def gather(i, b):
        return pltpu.async_copy(mp_hbm.at[idx_v.at[i]], rows_v.at[b], gsem[b])

    def scale(b, i):
        @plsc.parallel_loop(0, CHA // 16, step=1, unroll=SCALE_UNROLL)
        def sbody(g):
            wv = ew_v[i, pl.ds(g * 16, 16)]
            for l in range(16):
                j = g * 16 + l
                wb = jnp.full((16,), wv[l], jnp.float32)
                for q in range(nq):
                    rows_v[b, j, pl.ds(q * 16, 16)] = (
                        rows_v[b, j, pl.ds(q * 16, 16)] * wb)

    gh = [None, None]
    sh = [None, None]
    gh[0] = gather(0, 0)
    for i in range(ka):
        b = i % 2
        nb = (i + 1) % 2
        if i + 1 < ka:
            if sh[nb] is not None:
                sh[nb].wait()
            gh[nb] = gather(i + 1, nb)
        gh[b].wait()
        scale(b, i)
        sh[b] = pltpu.async_copy(rows_v.at[b], acc_sh.at[dst_v.at[i]],
                                 ssem[b], add=True)
    for h in sh:
        if h is not None:
            h.wait()
    plsc.subcore_barrier()

    for (o, w) in row_chunks:
        pltpu.sync_copy(acc_sh.at[pl.ds(s * npt + o, w)],
                        out_hbm.at[c, pl.ds(s * npt + o, w)])


def _make_agg_kernel(n, hid, ka):
    assert n % NS == 0 and hid % 16 == 0
    mesh = plsc.VectorSubcoreMesh(core_axis_name="c", subcore_axis_name="s")
    return pl.kernel(
        functools.partial(_agg_kernel_body, n),
        out_type=jax.ShapeDtypeStruct((NC, n, hid), jnp.float32),
        mesh=mesh,
        compiler_params=pltpu.CompilerParams(use_tc_tiling_on_sc=False),
        scratch_types=[
            pltpu.VMEM((ka, CHA), jnp.int32),
            pltpu.VMEM((ka, CHA), jnp.int32),
            pltpu.VMEM((ka, CHA), jnp.float32),
            pltpu.VMEM((2, CHA, hid), jnp.float32),
            pltpu.VMEM_SHARED((n, hid), jnp.float32),
            [pltpu.SemaphoreType.DMA, pltpu.SemaphoreType.DMA],
            [pltpu.SemaphoreType.DMA, pltpu.SemaphoreType.DMA],
        ],
    )


def _dinv(degp_ref):
    deg = 1.0 + degp_ref[0, :] + degp_ref[1, :]
    return lax.rsqrt(deg)


def _tc_a_body(x_ref, wf_ref, bf_ref, wc1_ref, degp_ref, mp_ref):
    h = jnp.maximum(
        jnp.dot(x_ref[...], wf_ref[...], preferred_element_type=jnp.float32)
        + bf_ref[...][None, :], 0.0)
    dinv = _dinv(degp_ref)
    mp_ref[...] = dinv[:, None] * jnp.dot(
        h, wc1_ref[...], preferred_element_type=jnp.float32)


def _tc_b_body(aggp_ref, mp_ref, b_ref, w_next_ref, degp_ref, out_ref):
    dinv = _dinv(degp_ref)[:, None]
    h = jnp.maximum(
        dinv * (aggp_ref[0] + aggp_ref[1] + mp_ref[...]) + b_ref[...][None, :],
        0.0)
    out_ref[...] = dinv * jnp.dot(
        h, w_next_ref[...], preferred_element_type=jnp.float32)


def _tc_c_body(aggp_ref, mp_ref, b_ref, wout_ref, bout_ref, degp_ref, out_ref):
    dinv = _dinv(degp_ref)[:, None]
    h = jnp.maximum(
        dinv * (aggp_ref[0] + aggp_ref[1] + mp_ref[...]) + b_ref[...][None, :],
        0.0)
    logits = jnp.dot(h, wout_ref[...], preferred_element_type=jnp.float32) \
        + bout_ref[...][None, :]
    m = jnp.max(logits, axis=-1, keepdims=True)
    lse = m + jnp.log(jnp.sum(jnp.exp(logits - m), axis=-1, keepdims=True))
    out_ref[...] = logits - lse


def _tc_call(body, out_shape):
    return pl.pallas_call(body, out_shape=jax.ShapeDtypeStruct(out_shape, jnp.float32))


def kernel(x, edge_index, edge_weight, W_first, b_first, W_c1, b_c1,
           W_c2, b_c2, W_out, b_out):
    n, f_in = x.shape
    hid = W_c1.shape[0]
    e = edge_weight.shape[0]
    assert e % (CHD * NW) == 0 and e % (CHA * NW) == 0
    kd = e // (CHD * NW)
    ka = e // (CHA * NW)
    src3a = edge_index[0].reshape(NW, ka, CHA)
    dst3a = edge_index[1].reshape(NW, ka, CHA)
    ew3a = edge_weight.reshape(NW, ka, CHA)
    dst3d = edge_index[1].reshape(NW, kd, CHD)
    ew3d = edge_weight.reshape(NW, kd, CHD)

    degf = _make_deg_kernel(n, kd)(dst3d, ew3d)
    degp = degf.reshape(NC, n)

    mp1 = _tc_call(_tc_a_body, (n, hid))(x, W_first, b_first, W_c1, degp)
    agg1 = _make_agg_kernel(n, hid, ka)(mp1, src3a, dst3a, ew3a)
    mp2 = _tc_call(_tc_b_body, (n, hid))(agg1, mp1, b_c1, W_c2, degp)
    agg2 = _make_agg_kernel(n, hid, ka)(mp2, src3a, dst3a, ew3a)
    out = _tc_call(_tc_c_body, (n, W_out.shape[1]))(
        agg2, mp2, b_c2, W_out, b_out, degp)
    return out
